# baseline (speedup 1.0000x reference)
import functools

import jax
import jax.numpy as jnp
import numpy as np
from jax import lax
from jax.experimental import pallas as pl
from jax.experimental.pallas import tpu as pltpu
from jax.experimental.pallas import tpu_sc as plsc

RATIO = 2

_NC = 2
_NS = 16

_IDX_CACHE = {}


def _sampled_idx(nums, index_num):
    if nums not in _IDX_CACHE:
        def _compute():
            perm = jax.random.permutation(jax.random.key(42), nums)
            return perm[:index_num].astype(jnp.int32)

        cpu = jax.local_devices(backend="cpu")[0]
        with jax.ensure_compile_time_eval(), jax.default_device(cpu):
            _IDX_CACHE[nums] = np.asarray(jax.jit(_compute)())
    return _IDX_CACHE[nums]


_PLAN_CACHE = {}


def _shuffle_plan(nums, index_num):
    if nums not in _PLAN_CACHE:
        idx = _sampled_idx(nums, index_num).astype(np.int64)
        seg = nums // _NS
        och = index_num // _NS
        k = np.arange(index_num, dtype=np.int64)
        t = idx // seg
        d = k // och
        key = t * _NS + d
        order = np.argsort(key, kind="stable")
        sizes = np.bincount(key, minlength=_NS * _NS)
        starts = np.concatenate(([0], np.cumsum(sizes)[:-1]))
        s_pad = int(-(-sizes.max() // 8) * 8)
        rank = k - starts[key[order]]
        ko = k[order]
        gather = np.zeros((_NS, _NS * s_pad), dtype=np.int32)
        gather[t[order], d[order] * s_pad + rank] = (idx[order] - t[order] * seg)
        perm = np.zeros((_NS, och), dtype=np.int32)
        perm[d[order], ko - d[order] * och] = t[order] * s_pad + rank
        _PLAN_CACHE[nums] = (gather, perm, s_pad)
    return _PLAN_CACHE[nums]


@functools.lru_cache(maxsize=None)
def _make_sampler(nums, c, index_num, s_pad):
    seg = nums // _NS
    och = index_num // _NS
    rpc = c // _NC
    blen = _NS * s_pad
    mesh = plsc.VectorSubcoreMesh(core_axis_name="c", subcore_axis_name="s")

    @functools.partial(
        pl.kernel,
        mesh=mesh,
        compiler_params=pltpu.CompilerParams(needs_layout_passes=False),
        out_type=jax.ShapeDtypeStruct((c, index_num), jnp.float32),
        scratch_types=[
            pltpu.VMEM_SHARED((_NS * blen,), jnp.float32),
            pltpu.VMEM_SHARED((_NS * blen,), jnp.float32),
            pltpu.VMEM((seg,), jnp.float32),
            pltpu.VMEM((seg,), jnp.float32),
            pltpu.VMEM((blen,), jnp.int32),
            pltpu.VMEM((och,), jnp.int32),
            pltpu.VMEM((blen,), jnp.float32),
            pltpu.VMEM((blen,), jnp.float32),
            pltpu.VMEM((och,), jnp.float32),
            pltpu.SemaphoreType.DMA,
            pltpu.SemaphoreType.DMA,
            pltpu.SemaphoreType.DMA,
        ],
    )
    def sample_kernel(x_hbm, g_hbm, q_hbm, out_hbm, shuf0, shuf1, seg0, seg1,
                      g_v, q_v, send_v, recv_v, out_v, ssem0, ssem1, wsem):
        cid = lax.axis_index("c")
        sid = lax.axis_index("s")
        shuf = (shuf0, shuf1)
        seg_v = (seg0, seg1)
        ssem = (ssem0, ssem1)
        pltpu.sync_copy(g_hbm.at[sid], g_v)
        pltpu.sync_copy(q_hbm.at[sid], q_v)

        def seg_src(r):
            return x_hbm.at[0, r, pl.ds(sid * seg, seg)]

        pltpu.async_copy(seg_src(cid * rpc), seg_v[0], ssem[0])

        @pl.loop(0, rpc, step=2)
        def _rows(i2):
            for b in (0, 1):
                r = cid * rpc + i2 + b
                pltpu.make_async_copy(seg_src(r), seg_v[b], ssem[b]).wait()
                pltpu.async_copy(seg_src(r + 1), seg_v[1 - b], ssem[1 - b])

                @pl.loop(0, blen // 16, unroll=8)
                def _phase1(j):
                    g = g_v[pl.ds(j * 16, 16)]
                    send_v[pl.ds(j * 16, 16)] = plsc.load_gather(
                        seg_v[b], [g]
                    )

                send_h = [
                    pltpu.async_copy(
                        send_v.at[pl.ds(dd * s_pad, s_pad)],
                        shuf[b].at[pl.ds((dd * _NS + sid) * s_pad, s_pad)],
                        wsem,
                    )
                    for dd in range(_NS)
                ]
                for h in send_h:
                    h.wait()
                plsc.subcore_barrier()
                pltpu.sync_copy(shuf[b].at[pl.ds(sid * blen, blen)], recv_v)
                plsc.subcore_barrier()

                @pl.loop(0, och // 16, unroll=8)
                def _phase2(j):
                    q = q_v[pl.ds(j * 16, 16)]
                    out_v[pl.ds(j * 16, 16)] = plsc.load_gather(recv_v, [q])

                pltpu.sync_copy(out_v, out_hbm.at[r, pl.ds(sid * och, och)])

        pltpu.make_async_copy(seg_src(0), seg_v[0], ssem[0]).wait()

    return sample_kernel


def kernel(x):
    b, c, nums = x.shape
    index_num = nums // RATIO
    gather, perm, s_pad = _shuffle_plan(nums, index_num)
    out = _make_sampler(nums, c, index_num, s_pad)(
        x, jnp.asarray(gather), jnp.asarray(perm)
    )
    return out.reshape(1, c, index_num)


# speedup vs baseline: 3.6717x; 3.6717x over previous
import functools

import jax
import jax.numpy as jnp
import numpy as np
from jax import lax
from jax.experimental import pallas as pl
from jax.experimental.pallas import tpu as pltpu
from jax.experimental.pallas import tpu_sc as plsc

RATIO = 2

_NC = 2
_NS = 16

_IDX_CACHE = {}


def _sampled_idx(nums, index_num):
    if nums not in _IDX_CACHE:
        def _compute():
            perm = jax.random.permutation(jax.random.key(42), nums)
            return perm[:index_num].astype(jnp.int32)

        cpu = jax.local_devices(backend="cpu")[0]
        with jax.ensure_compile_time_eval(), jax.default_device(cpu):
            _IDX_CACHE[nums] = np.asarray(jax.jit(_compute)())
    return _IDX_CACHE[nums]


@functools.lru_cache(maxsize=None)
def _make_sampler(nums, c, index_num):
    seg = nums // _NS
    och = index_num // _NS
    rpc = c // _NC
    mesh = plsc.VectorSubcoreMesh(core_axis_name="c", subcore_axis_name="s")

    @functools.partial(
        pl.kernel,
        mesh=mesh,
        out_type=jax.ShapeDtypeStruct((c, index_num), jnp.float32),
        scratch_types=[
            pltpu.VMEM_SHARED((nums,), jnp.float32),
            pltpu.VMEM_SHARED((nums,), jnp.float32),
            pltpu.VMEM_SHARED((nums,), jnp.float32),
            pltpu.VMEM((och,), jnp.int32),
            pltpu.VMEM((och,), jnp.float32),
            pltpu.VMEM((och,), jnp.float32),
            pltpu.SemaphoreType.DMA,
            pltpu.SemaphoreType.DMA,
            pltpu.SemaphoreType.DMA,
            pltpu.SemaphoreType.DMA,
            pltpu.SemaphoreType.DMA,
            pltpu.SemaphoreType.DMA,
        ],
    )
    def sample_kernel(xf_hbm, idx_hbm, out_hbm, row_sh0, row_sh1, row_sh2,
                      idx_v, out_v0, out_v1, gsem, ssem0, ssem1, ssem2,
                      osem0, osem1):
        cid = lax.axis_index("c")
        sid = lax.axis_index("s")
        row_sh = (row_sh0, row_sh1, row_sh2)
        out_v = (out_v0, out_v1)
        ssem = (ssem0, ssem1, ssem2)
        osem = (osem0, osem1)
        pltpu.sync_copy(idx_hbm.at[pl.ds(sid * och, och)], idx_v)

        def stage(i, b):
            r = cid * rpc + i
            return pltpu.async_copy(
                xf_hbm.at[0, r, pl.ds(sid * seg, seg)],
                row_sh[b].at[pl.ds(sid * seg, seg)],
                ssem[b],
            )

        stage_h = [stage(0, 0), None, None]
        out_h = [None, None]
        for i in range(rpc):
            b = i % 3
            bo = i % 2
            if i + 1 < rpc:
                stage_h[(i + 1) % 3] = stage(i + 1, (i + 1) % 3)
            stage_h[b].wait()
            plsc.subcore_barrier()
            if out_h[bo] is not None:
                out_h[bo].wait()
            gq = och // 4
            ghs = [
                pltpu.async_copy(
                    row_sh[b].at[idx_v.at[pl.ds(j * gq, gq)]],
                    out_v[bo].at[pl.ds(j * gq, gq)],
                    gsem,
                )
                for j in range(4)
            ]
            for gh in ghs:
                gh.wait()
            r = cid * rpc + i
            out_h[bo] = pltpu.async_copy(
                out_v[bo],
                out_hbm.at[r, pl.ds(sid * och, och)],
                osem[bo],
            )
        out_h[0].wait()
        out_h[1].wait()

    return sample_kernel


def kernel(x):
    b, c, nums = x.shape
    index_num = nums // RATIO
    idx = jnp.asarray(_sampled_idx(nums, index_num))
    out = _make_sampler(nums, c, index_num)(x, idx)
    return out.reshape(1, c, index_num)
